# Initial kernel scaffold; baseline (speedup 1.0000x reference)
import jax
import jax.numpy as jnp
from jax.experimental import pallas as pl


def kernel(neighbor_index, vertices, feature_map, directions, bias):
    raise NotImplementedError("write your pallas kernel here")


# speedup vs baseline: 3.6167x; 3.6167x over previous
import functools

import jax
import jax.numpy as jnp
from jax import lax
from jax.experimental import pallas as pl
from jax.experimental.pallas import tpu as pltpu

try:
    from jax.experimental.pallas import tpu_sc as plsc
except ImportError:
    plsc = None

_PAD = 128
_CHUNK = 128


def _sc_gather(table, gidx2d, num_rows):
    info = plsc.get_sparse_core_info()
    nw = info.num_cores * info.num_subcores
    rows_per_w = num_rows // nw
    chunks = rows_per_w // _CHUNK
    mesh = plsc.VectorSubcoreMesh(core_axis_name="c", subcore_axis_name="s")

    @functools.partial(
        pl.kernel,
        mesh=mesh,
        out_type=jax.ShapeDtypeStruct((num_rows, _PAD), jnp.float32),
        scratch_types=[
            pltpu.VMEM((chunks, _CHUNK), jnp.int32),
            pltpu.VMEM((_CHUNK, _PAD), jnp.float32),
            pltpu.VMEM((_CHUNK, _PAD), jnp.float32),
            pltpu.SemaphoreType.DMA,
            pltpu.SemaphoreType.DMA,
        ],
    )
    def k(table_hbm, idx_hbm, out_hbm, idx_v, buf0, buf1, sem0, sem1):
        wid = lax.axis_index("s") * info.num_cores + lax.axis_index("c")
        base = wid * rows_per_w
        pltpu.sync_copy(idx_hbm.at[pl.ds(wid * chunks, chunks)], idx_v)
        bufs = (buf0, buf1)
        sems = (sem0, sem1)
        copies = [None] * chunks
        for c in range(chunks):
            copies[c] = pltpu.async_copy(
                table_hbm.at[idx_v.at[c]], bufs[c % 2], sems[c % 2])
            if c > 0:
                copies[c - 1].wait()
                pltpu.sync_copy(
                    bufs[(c - 1) % 2],
                    out_hbm.at[pl.ds(base + (c - 1) * _CHUNK, _CHUNK)])
        copies[chunks - 1].wait()
        pltpu.sync_copy(
            bufs[(chunks - 1) % 2],
            out_hbm.at[pl.ds(base + (chunks - 1) * _CHUNK, _CHUNK)])

    return k(table, gidx2d)


def _rsqrt_refined(s):
    r = lax.rsqrt(jnp.maximum(s, 1e-30))
    r = r * (1.5 - 0.5 * s * r * r)
    return jnp.where(s > 0.0, r, 0.0)


def _tc_body(nbr_ref, vert_ref, fm_ref, sup_ref, bias_ref, out_ref):
    sup = sup_ref[...]
    s2 = sup[0:1] * sup[0:1] + sup[1:2] * sup[1:2] + sup[2:3] * sup[2:3]
    sinv = _rsqrt_refined(s2)
    sx = sup[0:1] * sinv
    sy = sup[1:2] * sinv
    sz = sup[2:3] * sinv

    vert = vert_ref[...]
    n = nbr_ref.shape[0]
    m = None
    for j in range(n):
        d = nbr_ref[j] - vert
        dx = d[:, 0:1]
        dy = d[:, 1:2]
        dz = d[:, 2:3]
        s = dx * dx + dy * dy + dz * dz
        inv = _rsqrt_refined(s)
        th = (dx * inv) * sx + (dy * inv) * sy + (dz * inv) * sz
        m = th if m is None else jnp.maximum(m, th)
    m = jnp.maximum(m, 0.0)

    fm = fm_ref[...]
    o = fm.shape[1]
    s_num = m.shape[1] // o
    cols = [
        jnp.sum(fm * m[:, si * o:(si + 1) * o], axis=1, keepdims=True)
        for si in range(s_num)
    ]
    out_ref[...] = jnp.concatenate(cols, axis=1) + bias_ref[...]


def _tc_compute(nbr3, vert_pad, fm, sup_pad, bias2d, rows_block=128):
    n, bv, _ = nbr3.shape
    so = sup_pad.shape[1]
    s_num = bias2d.shape[1]
    grid = (bv // rows_block,)
    return pl.pallas_call(
        _tc_body,
        grid=grid,
        in_specs=[
            pl.BlockSpec((n, rows_block, _PAD), lambda i: (0, i, 0)),
            pl.BlockSpec((rows_block, _PAD), lambda i: (i, 0)),
            pl.BlockSpec((rows_block, fm.shape[1]), lambda i: (i, 0)),
            pl.BlockSpec((8, so), lambda i: (0, 0)),
            pl.BlockSpec((1, s_num), lambda i: (0, 0)),
        ],
        out_specs=pl.BlockSpec((rows_block, s_num), lambda i: (i, 0)),
        out_shape=jax.ShapeDtypeStruct((bv, s_num), jnp.float32),
    )(nbr3, vert_pad, fm, sup_pad, bias2d)


def kernel(neighbor_index, vertices, feature_map, directions, bias):
    bs, v, n = neighbor_index.shape
    o = feature_map.shape[-1]
    s_num = directions.shape[1] // o
    bv = bs * v

    idx = neighbor_index.astype(jnp.int32)
    gidx = idx + (jnp.arange(bs, dtype=jnp.int32) * v)[:, None, None]
    gidx_j = jnp.transpose(gidx, (2, 0, 1)).reshape(-1, _CHUNK)

    table = jnp.zeros((bv, _PAD), jnp.float32).at[:, :3].set(
        vertices.reshape(bv, 3))
    nbr = _sc_gather(table, gidx_j, n * bv)
    nbr3 = nbr.reshape(n, bv, _PAD)

    fm = feature_map.reshape(bv, o)
    sup_pad = jnp.zeros((8, s_num * o), jnp.float32).at[:3].set(directions)
    bias2d = bias.reshape(1, s_num)

    out = _tc_compute(nbr3, table, fm, sup_pad, bias2d)
    return out.reshape(bs, v, s_num)


# speedup vs baseline: 7.8148x; 2.1607x over previous
import functools

import jax
import jax.numpy as jnp
from jax import lax
from jax.experimental import pallas as pl
from jax.experimental.pallas import tpu as pltpu

try:
    from jax.experimental.pallas import tpu_sc as plsc
except ImportError:
    plsc = None

_L = 16


def _sc_dirs(xs, ys, zs, gidx_j, num_rows):
    info = plsc.get_sparse_core_info()
    nw = info.num_cores * info.num_subcores
    per_w = num_rows // nw
    steps = per_w // _L
    bv = xs.shape[0]
    mesh = plsc.VectorSubcoreMesh(core_axis_name="c", subcore_axis_name="s")
    plane = jax.ShapeDtypeStruct((num_rows,), jnp.float32)

    @functools.partial(
        pl.kernel,
        mesh=mesh,
        out_type=(plane, plane, plane),
        compiler_params=pltpu.CompilerParams(needs_layout_passes=False),
        scratch_types=[
            pltpu.VMEM((bv,), jnp.float32),
            pltpu.VMEM((bv,), jnp.float32),
            pltpu.VMEM((bv,), jnp.float32),
            pltpu.VMEM((per_w,), jnp.int32),
            pltpu.VMEM((per_w,), jnp.float32),
            pltpu.VMEM((per_w,), jnp.float32),
            pltpu.VMEM((per_w,), jnp.float32),
        ],
    )
    def k(xs_hbm, ys_hbm, zs_hbm, idx_hbm, ox_hbm, oy_hbm, oz_hbm,
          xs_v, ys_v, zs_v, idx_v, dx_v, dy_v, dz_v):
        wid = lax.axis_index("s") * info.num_cores + lax.axis_index("c")
        base = wid * per_w
        gbase = lax.rem(base, bv)
        pltpu.sync_copy(xs_hbm, xs_v)
        pltpu.sync_copy(ys_hbm, ys_v)
        pltpu.sync_copy(zs_hbm, zs_v)
        pltpu.sync_copy(idx_hbm.at[pl.ds(base, per_w)], idx_v)

        def body(i, _):
            off = i * _L
            iv = idx_v[pl.ds(off, _L)]
            gx = plsc.load_gather(xs_v, [iv])
            gy = plsc.load_gather(ys_v, [iv])
            gz = plsc.load_gather(zs_v, [iv])
            coff = gbase + off
            dx_v[pl.ds(off, _L)] = gx - xs_v[pl.ds(coff, _L)]
            dy_v[pl.ds(off, _L)] = gy - ys_v[pl.ds(coff, _L)]
            dz_v[pl.ds(off, _L)] = gz - zs_v[pl.ds(coff, _L)]
            return 0

        lax.fori_loop(0, steps, body, 0)
        pltpu.sync_copy(dx_v, ox_hbm.at[pl.ds(base, per_w)])
        pltpu.sync_copy(dy_v, oy_hbm.at[pl.ds(base, per_w)])
        pltpu.sync_copy(dz_v, oz_hbm.at[pl.ds(base, per_w)])

    return k(xs, ys, zs, gidx_j)


def _rsqrt_refined(s):
    r = lax.rsqrt(jnp.maximum(s, 1e-30))
    r = r * (1.5 - 0.5 * s * r * r)
    return jnp.where(s > 0.0, r, 0.0)


def _tc_body(x_ref, y_ref, z_ref, fmt_ref, supt_ref, bias_ref, out_ref):
    supt = supt_ref[...]
    sxc = supt[:, 0:1]
    syc = supt[:, 1:2]
    szc = supt[:, 2:3]
    s2 = sxc * sxc + syc * syc + szc * szc
    sinv = _rsqrt_refined(s2)
    sxc = sxc * sinv
    syc = syc * sinv
    szc = szc * sinv

    x = x_ref[...]
    y = y_ref[...]
    z = z_ref[...]
    s = x * x + y * y + z * z
    inv = _rsqrt_refined(s)
    xn = x * inv
    yn = y * inv
    zn = z * inv

    fmt = fmt_ref[...]
    n = x.shape[0]
    o = fmt.shape[0]
    s_num = supt.shape[0] // o
    for t in range(s_num):
        sx_t = sxc[t * o:(t + 1) * o]
        sy_t = syc[t * o:(t + 1) * o]
        sz_t = szc[t * o:(t + 1) * o]
        m = None
        for j in range(n):
            th = xn[j:j + 1] * sx_t + yn[j:j + 1] * sy_t + zn[j:j + 1] * sz_t
            m = th if m is None else jnp.maximum(m, th)
        m = jnp.maximum(m, 0.0)
        col = jnp.sum(fmt * m, axis=0, keepdims=True)
        out_ref[t:t + 1, :] = col + bias_ref[t:t + 1, 0:1]


def _tc_compute(xt, yt, zt, fmt, supt_pad, bias2d, rows_block=128):
    n, bv = xt.shape
    o = fmt.shape[0]
    s_num = bias2d.shape[0]
    dir_spec = pl.BlockSpec((n, rows_block), lambda i: (0, i))
    return pl.pallas_call(
        _tc_body,
        grid=(bv // rows_block,),
        in_specs=[
            dir_spec,
            dir_spec,
            dir_spec,
            pl.BlockSpec((o, rows_block), lambda i: (0, i)),
            pl.BlockSpec((s_num * o, 8), lambda i: (0, 0)),
            pl.BlockSpec((s_num, 1), lambda i: (0, 0)),
        ],
        out_specs=pl.BlockSpec((s_num, rows_block), lambda i: (0, i)),
        out_shape=jax.ShapeDtypeStruct((s_num, bv), jnp.float32),
    )(xt, yt, zt, fmt, supt_pad, bias2d)


def kernel(neighbor_index, vertices, feature_map, directions, bias):
    bs, v, n = neighbor_index.shape
    o = feature_map.shape[-1]
    s_num = directions.shape[1] // o
    bv = bs * v

    idx = neighbor_index.astype(jnp.int32)
    gidx = idx + (jnp.arange(bs, dtype=jnp.int32) * v)[:, None, None]
    gidx_j = jnp.transpose(gidx, (2, 0, 1)).reshape(-1)

    vflat = vertices.reshape(bv, 3)
    dx, dy, dz = _sc_dirs(vflat[:, 0], vflat[:, 1], vflat[:, 2],
                          gidx_j, n * bv)

    fmt = feature_map.reshape(bv, o).T
    supt_pad = jnp.zeros((s_num * o, 8), jnp.float32).at[:, :3].set(
        directions.T)
    bias2d = bias.reshape(s_num, 1)

    out_t = _tc_compute(dx.reshape(n, bv), dy.reshape(n, bv),
                        dz.reshape(n, bv), fmt, supt_pad, bias2d)
    return out_t.T.reshape(bs, v, s_num)


# speedup vs baseline: 8.7326x; 1.1174x over previous
import functools

import jax
import jax.numpy as jnp
from jax import lax
from jax.experimental import pallas as pl
from jax.experimental.pallas import tpu as pltpu

try:
    from jax.experimental.pallas import tpu_sc as plsc
except ImportError:
    plsc = None

_L = 16


def _sc_dirs(xs, ys, zs, gidx_j, num_rows):
    info = plsc.get_sparse_core_info()
    nw = info.num_cores * info.num_subcores
    per_w = num_rows // nw
    steps = per_w // _L
    bv = xs.shape[0]
    mesh = plsc.VectorSubcoreMesh(core_axis_name="c", subcore_axis_name="s")
    plane = jax.ShapeDtypeStruct((num_rows,), jnp.float32)

    @functools.partial(
        pl.kernel,
        mesh=mesh,
        out_type=(plane, plane, plane),
        compiler_params=pltpu.CompilerParams(needs_layout_passes=False),
        scratch_types=[
            pltpu.VMEM((bv,), jnp.float32),
            pltpu.VMEM((bv,), jnp.float32),
            pltpu.VMEM((bv,), jnp.float32),
            pltpu.VMEM((per_w,), jnp.int32),
            pltpu.VMEM((per_w,), jnp.float32),
            pltpu.VMEM((per_w,), jnp.float32),
            pltpu.VMEM((per_w,), jnp.float32),
        ],
    )
    def k(xs_hbm, ys_hbm, zs_hbm, idx_hbm, ox_hbm, oy_hbm, oz_hbm,
          xs_v, ys_v, zs_v, idx_v, dx_v, dy_v, dz_v):
        wid = lax.axis_index("s") * info.num_cores + lax.axis_index("c")
        base = wid * per_w
        gbase = lax.rem(base, bv)
        pltpu.sync_copy(xs_hbm, xs_v)
        pltpu.sync_copy(ys_hbm, ys_v)
        pltpu.sync_copy(zs_hbm, zs_v)
        pltpu.sync_copy(idx_hbm.at[pl.ds(base, per_w)], idx_v)

        def body(i, _):
            off = i * _L
            iv = idx_v[pl.ds(off, _L)]
            gx = plsc.load_gather(xs_v, [iv])
            gy = plsc.load_gather(ys_v, [iv])
            gz = plsc.load_gather(zs_v, [iv])
            coff = gbase + off
            dx_v[pl.ds(off, _L)] = gx - xs_v[pl.ds(coff, _L)]
            dy_v[pl.ds(off, _L)] = gy - ys_v[pl.ds(coff, _L)]
            dz_v[pl.ds(off, _L)] = gz - zs_v[pl.ds(coff, _L)]
            return 0

        lax.fori_loop(0, steps, body, 0)
        pltpu.sync_copy(dx_v, ox_hbm.at[pl.ds(base, per_w)])
        pltpu.sync_copy(dy_v, oy_hbm.at[pl.ds(base, per_w)])
        pltpu.sync_copy(dz_v, oz_hbm.at[pl.ds(base, per_w)])

    return k(xs, ys, zs, gidx_j)


def _rsqrt_refined(s):
    r = lax.rsqrt(jnp.maximum(s, 1e-30))
    r = r * (1.5 - 0.5 * s * r * r)
    return jnp.where(s > 0.0, r, 0.0)


def _tc_body(x_ref, y_ref, z_ref, fm_ref, supt_ref, bias_ref, out_ref,
             sup_scr):
    @pl.when(pl.program_id(0) == 0)
    def _():
        supt = supt_ref[...]
        sxc = supt[:, 0:1]
        syc = supt[:, 1:2]
        szc = supt[:, 2:3]
        s2 = sxc * sxc + syc * syc + szc * szc
        sinv = _rsqrt_refined(s2)
        sup_scr[:, 0:1] = sxc * sinv
        sup_scr[:, 1:2] = syc * sinv
        sup_scr[:, 2:3] = szc * sinv

    x = x_ref[...]
    y = y_ref[...]
    z = z_ref[...]
    s = x * x + y * y + z * z
    inv = _rsqrt_refined(s)
    xn = x * inv
    yn = y * inv
    zn = z * inv

    fmt = fm_ref[...].T
    n = x.shape[0]
    o = fmt.shape[0]
    s_num = supt_ref.shape[0] // o
    cols = []
    for t in range(s_num):
        sx_t = sup_scr[t * o:(t + 1) * o, 0:1]
        sy_t = sup_scr[t * o:(t + 1) * o, 1:2]
        sz_t = sup_scr[t * o:(t + 1) * o, 2:3]
        m = None
        for j in range(n):
            th = xn[j:j + 1] * sx_t + yn[j:j + 1] * sy_t + zn[j:j + 1] * sz_t
            m = th if m is None else jnp.maximum(m, th)
        m = jnp.maximum(m, 0.0)
        cols.append(jnp.sum(fmt * m, axis=0, keepdims=True))
    out_t = jnp.concatenate(cols, axis=0)
    out_ref[...] = out_t.T + bias_ref[...]


def _tc_compute(xt, yt, zt, fm, supt_pad, bias2d, rows_block=128):
    n, bv = xt.shape
    o = fm.shape[1]
    s_num = bias2d.shape[1]
    dir_spec = pl.BlockSpec((n, rows_block), lambda i: (0, i))
    return pl.pallas_call(
        _tc_body,
        grid=(bv // rows_block,),
        in_specs=[
            dir_spec,
            dir_spec,
            dir_spec,
            pl.BlockSpec((rows_block, o), lambda i: (i, 0)),
            pl.BlockSpec((s_num * o, 8), lambda i: (0, 0)),
            pl.BlockSpec((1, s_num), lambda i: (0, 0)),
        ],
        out_specs=pl.BlockSpec((rows_block, s_num), lambda i: (i, 0)),
        out_shape=jax.ShapeDtypeStruct((bv, s_num), jnp.float32),
        scratch_shapes=[pltpu.VMEM((s_num * o, 8), jnp.float32)],
    )(xt, yt, zt, fm, supt_pad, bias2d)


def kernel(neighbor_index, vertices, feature_map, directions, bias):
    bs, v, n = neighbor_index.shape
    o = feature_map.shape[-1]
    s_num = directions.shape[1] // o
    bv = bs * v

    idx = neighbor_index.astype(jnp.int32)
    gidx = idx + (jnp.arange(bs, dtype=jnp.int32) * v)[:, None, None]
    gidx_j = jnp.transpose(gidx, (2, 0, 1)).reshape(-1)

    vflat = vertices.reshape(bv, 3)
    dx, dy, dz = _sc_dirs(vflat[:, 0], vflat[:, 1], vflat[:, 2],
                          gidx_j, n * bv)

    fm = feature_map.reshape(bv, o)
    supt_pad = jnp.zeros((s_num * o, 8), jnp.float32).at[:, :3].set(
        directions.T)
    bias2d = bias.reshape(1, s_num)

    out = _tc_compute(dx.reshape(n, bv), dy.reshape(n, bv),
                      dz.reshape(n, bv), fm, supt_pad, bias2d)
    return out.reshape(bs, v, s_num)
